# Initial kernel scaffold; baseline (speedup 1.0000x reference)
import jax
import jax.numpy as jnp
from jax.experimental import pallas as pl


def kernel(H0, Z0, beta, batch_ids, edge_index, W_embed, b_embed, Wm1, bm1, Wm2, bm2, Wx, Wu, bu, W1, b1, W2, b2):
    raise NotImplementedError("write your pallas kernel here")


# speedup vs baseline: 2.6382x; 2.6382x over previous
import functools

import jax
import jax.numpy as jnp
from jax import lax
from jax.experimental import pallas as pl
from jax.experimental.pallas import tpu as pltpu
from jax.experimental.pallas import tpu_sc as plsc

N = 10000
E = 320000
HID = 128
NB = 2000
EB = 2000
CH = 128
NCH = E // CH
NC = 2
NS = 16
NW = NC * NS
GLOOPS = -(-NCH // NW)
SLOOPS = -(-NCH // NS)
N_PAD = 10240
ROWS_PER_TILE = N_PAD // NS


def _silu(v):
    return v * jax.nn.sigmoid(v)


def _embed_body(h0_ref, beta_ref, wea_ref, we8_ref, we9_ref,
                we10_ref, be_ref, out_ref):
    b = beta_ref[...]
    h = jnp.dot(h0_ref[...], wea_ref[...], preferred_element_type=jnp.float32)
    out_ref[...] = h + b * we8_ref[...] + jnp.sin(b) * we9_ref[...] \
        + jnp.cos(b) * we10_ref[...] + be_ref[...]


def _embed(H0, beta2, WeA, we8, we9, we10, be_row):
    return pl.pallas_call(
        _embed_body,
        grid=(N // NB,),
        in_specs=[
            pl.BlockSpec((NB, 8), lambda i: (i, 0)),
            pl.BlockSpec((NB, 1), lambda i: (i, 0)),
            pl.BlockSpec((8, HID), lambda i: (0, 0)),
            pl.BlockSpec((1, HID), lambda i: (0, 0)),
            pl.BlockSpec((1, HID), lambda i: (0, 0)),
            pl.BlockSpec((1, HID), lambda i: (0, 0)),
            pl.BlockSpec((1, HID), lambda i: (0, 0)),
        ],
        out_specs=pl.BlockSpec((NB, HID), lambda i: (i, 0)),
        out_shape=jax.ShapeDtypeStruct((N, HID), jnp.float32),
    )(H0, beta2, WeA, we8, we9, we10, be_row)


def _sc_mesh():
    return plsc.VectorSubcoreMesh(core_axis_name="c", subcore_axis_name="s")


def _gather_kernel(th_hbm, tx_hbm, src_hbm, dst_hbm,
                   hsh_out, hsx_out, hdh_out, hdx_out,
                   idx_v, rh_v, rx_v, sem):
    c = lax.axis_index("c")
    s = lax.axis_index("s")
    wid = s * NC + c

    def one_side(idx_hbm, h_out, x_out, base):
        pltpu.sync_copy(idx_hbm.at[pl.ds(base, CH)], idx_v)
        pltpu.async_copy(th_hbm.at[idx_v], rh_v, sem).wait()
        pltpu.sync_copy(rh_v, h_out.at[pl.ds(base, CH)])
        pltpu.async_copy(tx_hbm.at[idx_v], rx_v, sem).wait()
        pltpu.sync_copy(rx_v, x_out.at[pl.ds(base, CH)])

    def body(i, carry):
        ch = wid + i * NW

        @pl.when(ch < NCH)
        def _():
            base = ch * CH
            one_side(src_hbm, hsh_out, hsx_out, base)
            one_side(dst_hbm, hdh_out, hdx_out, base)

        return carry

    lax.fori_loop(0, GLOOPS, body, 0)


def _gather(Th, Tx, src, dst):
    out4 = [jax.ShapeDtypeStruct((E, HID), jnp.float32)] * 4
    fn = functools.partial(
        pl.kernel,
        mesh=_sc_mesh(),
        out_type=out4,
        scratch_types=[
            pltpu.VMEM((CH,), jnp.int32),
            pltpu.VMEM((CH, HID), jnp.float32),
            pltpu.VMEM((CH, HID), jnp.float32),
            pltpu.SemaphoreType.DMA,
        ],
    )(_gather_kernel)
    return fn(Th, Tx, src, dst)


def _edge_body(hsh_ref, hsx_ref, hdh_ref, hdx_ref, wa_ref, wb_ref, wr2_ref,
               b1_ref, w2_ref, b2_ref, wx_ref, mh_ref, mx_ref):
    dx = hsx_ref[...] - hdx_ref[...]
    r2 = jnp.sum(dx * dx, axis=1, keepdims=True)
    m = jnp.dot(hsh_ref[...], wa_ref[...], preferred_element_type=jnp.float32)
    m = m + jnp.dot(hdh_ref[...], wb_ref[...], preferred_element_type=jnp.float32)
    m = _silu(m + r2 * wr2_ref[...] + b1_ref[...])
    m = _silu(jnp.dot(m, w2_ref[...], preferred_element_type=jnp.float32)
              + b2_ref[...])
    coef = jnp.sum(m * wx_ref[...], axis=1, keepdims=True)
    lane = lax.broadcasted_iota(jnp.int32, (EB, HID), 1)
    mh_ref[...] = m
    mx_ref[...] = dx * coef + (lane == 3).astype(jnp.float32)


def _edge(HSH, HSX, HDH, HDX, WA, WB, wr2, b1r, W2, b2r, wxr):
    full = lambda i: (0, 0)
    edge_spec = pl.BlockSpec((EB, HID), lambda i: (i, 0))
    return pl.pallas_call(
        _edge_body,
        grid=(E // EB,),
        in_specs=[
            edge_spec, edge_spec, edge_spec, edge_spec,
            pl.BlockSpec((HID, HID), full),
            pl.BlockSpec((HID, HID), full),
            pl.BlockSpec((1, HID), full),
            pl.BlockSpec((1, HID), full),
            pl.BlockSpec((HID, HID), full),
            pl.BlockSpec((1, HID), full),
            pl.BlockSpec((1, HID), full),
        ],
        out_specs=[edge_spec, edge_spec],
        out_shape=[jax.ShapeDtypeStruct((E, HID), jnp.float32)] * 2,
    )(HSH, HSX, HDH, HDX, WA, WB, wr2, b1r, W2, b2r, wxr)


def _scatter_kernel(mh_hbm, mx_hbm, dst_hbm, zeros_hbm, acc_out,
                    idx_v, rows_v, acc_sh, sem):
    c = lax.axis_index("c")
    s = lax.axis_index("s")
    pltpu.sync_copy(zeros_hbm,
                    acc_sh.at[pl.ds(s * ROWS_PER_TILE, ROWS_PER_TILE)])
    plsc.subcore_barrier()

    def add_chunks(m_hbm):
        def body(i, carry):
            ch = s + i * NS

            @pl.when(ch < NCH)
            def _():
                base = ch * CH
                pltpu.sync_copy(dst_hbm.at[pl.ds(base, CH)], idx_v)
                pltpu.sync_copy(m_hbm.at[pl.ds(base, CH)], rows_v)
                pltpu.sync_copy(rows_v, acc_sh.at[idx_v], add=True)

            return carry

        lax.fori_loop(0, SLOOPS, body, 0)

    @pl.when(c == 0)
    def _():
        add_chunks(mh_hbm)

    @pl.when(c == 1)
    def _():
        add_chunks(mx_hbm)

    plsc.subcore_barrier()
    pltpu.sync_copy(acc_sh.at[pl.ds(s * ROWS_PER_TILE, ROWS_PER_TILE)],
                    acc_out.at[c, pl.ds(s * ROWS_PER_TILE, ROWS_PER_TILE)])


def _scatter(MH, MX, dst, zeros_tile):
    fn = functools.partial(
        pl.kernel,
        mesh=_sc_mesh(),
        out_type=jax.ShapeDtypeStruct((NC, N_PAD, HID), jnp.float32),
        scratch_types=[
            pltpu.VMEM((CH,), jnp.int32),
            pltpu.VMEM((CH, HID), jnp.float32),
            pltpu.VMEM_SHARED((N_PAD, HID), jnp.float32),
            pltpu.SemaphoreType.DMA,
        ],
    )(_scatter_kernel)
    return fn(MH, MX, dst, zeros_tile)


def _node_body(th_ref, tx_ref, acc_ref, wua_ref, wub_ref, bu_ref,
               oh_ref, ox_ref):
    h = th_ref[...]
    x = tx_ref[...]
    agg = acc_ref[0]
    xa = acc_ref[1]
    lane = lax.broadcasted_iota(jnp.int32, (NB, HID), 1)
    deg = jnp.sum(jnp.where(lane == 3, xa, 0.0), axis=1, keepdims=True) + 1.0
    u = jnp.dot(h, wua_ref[...], preferred_element_type=jnp.float32)
    u = u + jnp.dot(agg, wub_ref[...], preferred_element_type=jnp.float32)
    oh_ref[...] = h + _silu(u + bu_ref[...])
    ox_ref[...] = x + jnp.where(lane < 3, xa, 0.0) / deg


def _node(Th, Tx, ACC, WuA, WuB, bur):
    full = lambda i: (0, 0)
    node_spec = pl.BlockSpec((NB, HID), lambda i: (i, 0))
    return pl.pallas_call(
        _node_body,
        grid=(N // NB,),
        in_specs=[
            node_spec, node_spec,
            pl.BlockSpec((NC, NB, HID), lambda i: (0, i, 0)),
            pl.BlockSpec((HID, HID), full),
            pl.BlockSpec((HID, HID), full),
            pl.BlockSpec((1, HID), full),
        ],
        out_specs=[node_spec, node_spec],
        out_shape=[jax.ShapeDtypeStruct((N, HID), jnp.float32)] * 2,
    )(Th, Tx, ACC, WuA, WuB, bur)


def _pool_body(th_ref, bid_ref, w1_ref, b1_ref, w2_ref, b2_ref, out_ref):
    bid = bid_ref[...]
    row = lax.broadcasted_iota(jnp.int32, (64, N), 0)
    oht = jnp.where(row == bid, 1.0, 0.0)
    val = jnp.dot(oht, th_ref[...], preferred_element_type=jnp.float32)
    n = jnp.sum(oht, axis=1, keepdims=True)
    val = val * lax.rsqrt(jnp.maximum(n, 1.0))
    o = jnp.dot(_silu(val), w1_ref[...], preferred_element_type=jnp.float32) \
        + b1_ref[...]
    o = jnp.dot(_silu(o), w2_ref[...], preferred_element_type=jnp.float32) \
        + b2_ref[...]
    out_ref[...] = o


def _pool(Th, bid_row, W1, b1r, W2, b2r):
    return pl.pallas_call(
        _pool_body,
        in_specs=[
            pl.BlockSpec((N, HID), lambda: (0, 0)),
            pl.BlockSpec((1, N), lambda: (0, 0)),
            pl.BlockSpec((HID, HID), lambda: (0, 0)),
            pl.BlockSpec((1, HID), lambda: (0, 0)),
            pl.BlockSpec((HID, 128), lambda: (0, 0)),
            pl.BlockSpec((1, 128), lambda: (0, 0)),
        ],
        out_specs=pl.BlockSpec((64, 128), lambda: (0, 0)),
        out_shape=jax.ShapeDtypeStruct((64, 128), jnp.float32),
    )(Th, bid_row, W1, b1r, W2, b2r)


def kernel(H0, Z0, beta, batch_ids, edge_index, W_embed, b_embed, Wm1, bm1,
           Wm2, bm2, Wx, Wu, bu, W1, b1, W2, b2):
    f32 = jnp.float32
    src = edge_index[0].astype(jnp.int32)
    dst = edge_index[1].astype(jnp.int32)
    bid_row = batch_ids.astype(jnp.int32).reshape(1, N)
    beta2 = beta.reshape(N, 1)
    zeros_tile = jnp.zeros((ROWS_PER_TILE, HID), f32)

    Th = _embed(H0, beta2, W_embed[:8], W_embed[8:9], W_embed[9:10],
                W_embed[10:11], b_embed.reshape(1, HID))
    Tx = jnp.concatenate([Z0[:, 0, :], jnp.zeros((N, HID - 3), f32)], axis=1)

    for l in range(3):
        HSH, HSX, HDH, HDX = _gather(Th, Tx, src, dst)
        MH, MX = _edge(HSH, HSX, HDH, HDX,
                       Wm1[l, :HID], Wm1[l, HID:2 * HID],
                       Wm1[l, 2 * HID:2 * HID + 1],
                       bm1[l].reshape(1, HID), Wm2[l], bm2[l].reshape(1, HID),
                       Wx[l, :, 0].reshape(1, HID))
        ACC = _scatter(MH, MX, dst, zeros_tile)
        Th, Tx = _node(Th, Tx, ACC, Wu[l, :HID], Wu[l, HID:],
                       bu[l].reshape(1, HID))

    return _pool(Th, bid_row, W1, b1.reshape(1, HID), W2, b2.reshape(1, 128))


# speedup vs baseline: 3.5219x; 1.3350x over previous
import functools

import jax
import jax.numpy as jnp
from jax import lax
from jax.experimental import pallas as pl
from jax.experimental.pallas import tpu as pltpu
from jax.experimental.pallas import tpu_sc as plsc

N = 10000
E = 320000
HID = 128
NB = 2000
EB = 2000
CH = 128
NCH = E // CH
NC = 2
NS = 16
NW = NC * NS
GLOOPS = -(-NCH // NW)
SLOOPS = -(-NCH // NS)
N_PAD = 10240
ROWS_PER_TILE = N_PAD // NS


def _silu(v):
    return v * jax.nn.sigmoid(v)


def _pack(h, x):
    hb = lax.bitcast_convert_type(h.astype(jnp.bfloat16).astype(jnp.float32),
                                  jnp.int32)
    xb = lax.bitcast_convert_type(x.astype(jnp.bfloat16).astype(jnp.float32),
                                  jnp.int32)
    return jnp.bitwise_or(hb, lax.shift_right_logical(xb, 16))


def _unpack_hi(p):
    return lax.bitcast_convert_type(
        jnp.bitwise_and(p, jnp.int32(-65536)), jnp.float32)


def _unpack_lo(p):
    return lax.bitcast_convert_type(lax.shift_left(p, 16), jnp.float32)


def _embed_body(h0_ref, beta_ref, z_ref, wea_ref, we8_ref, we9_ref,
                we10_ref, be_ref, out_ref, obf_ref):
    b = beta_ref[...]
    h = jnp.dot(h0_ref[...], wea_ref[...], preferred_element_type=jnp.float32)
    h = h + b * we8_ref[...] + jnp.sin(b) * we9_ref[...] \
        + jnp.cos(b) * we10_ref[...] + be_ref[...]
    out_ref[...] = h
    x = jnp.concatenate([z_ref[...], jnp.zeros((NB, HID - 16), jnp.float32)],
                        axis=1)
    obf_ref[...] = _pack(h, x)


def _embed(H0, beta2, z16, WeA, we8, we9, we10, be_row):
    return pl.pallas_call(
        _embed_body,
        grid=(N // NB,),
        in_specs=[
            pl.BlockSpec((NB, 8), lambda i: (i, 0)),
            pl.BlockSpec((NB, 1), lambda i: (i, 0)),
            pl.BlockSpec((NB, 16), lambda i: (i, 0)),
            pl.BlockSpec((8, HID), lambda i: (0, 0)),
            pl.BlockSpec((1, HID), lambda i: (0, 0)),
            pl.BlockSpec((1, HID), lambda i: (0, 0)),
            pl.BlockSpec((1, HID), lambda i: (0, 0)),
            pl.BlockSpec((1, HID), lambda i: (0, 0)),
        ],
        out_specs=[
            pl.BlockSpec((NB, HID), lambda i: (i, 0)),
            pl.BlockSpec((NB, HID), lambda i: (i, 0)),
        ],
        out_shape=[
            jax.ShapeDtypeStruct((N, HID), jnp.float32),
            jax.ShapeDtypeStruct((N, HID), jnp.int32),
        ],
    )(H0, beta2, z16, WeA, we8, we9, we10, be_row)


def _sc_mesh():
    return plsc.VectorSubcoreMesh(core_axis_name="c", subcore_axis_name="s")


def _gather_kernel(tbf_hbm, src_hbm, dst_hbm, hs_out, hd_out,
                   idx_v, rows_v, sem):
    c = lax.axis_index("c")
    s = lax.axis_index("s")
    wid = s * NC + c

    def body(i, carry):
        ch = wid + i * NW

        @pl.when(ch < NCH)
        def _():
            base = ch * CH
            pltpu.sync_copy(src_hbm.at[pl.ds(base, CH)], idx_v)
            pltpu.async_copy(tbf_hbm.at[idx_v], rows_v, sem).wait()
            pltpu.sync_copy(rows_v, hs_out.at[pl.ds(base, CH)])
            pltpu.sync_copy(dst_hbm.at[pl.ds(base, CH)], idx_v)
            pltpu.async_copy(tbf_hbm.at[idx_v], rows_v, sem).wait()
            pltpu.sync_copy(rows_v, hd_out.at[pl.ds(base, CH)])

        return carry

    lax.fori_loop(0, GLOOPS, body, 0)


def _gather(Tbf, src, dst):
    fn = functools.partial(
        pl.kernel,
        mesh=_sc_mesh(),
        out_type=[jax.ShapeDtypeStruct((E, HID), jnp.int32)] * 2,
        scratch_types=[
            pltpu.VMEM((CH,), jnp.int32),
            pltpu.VMEM((CH, HID), jnp.int32),
            pltpu.SemaphoreType.DMA,
        ],
    )(_gather_kernel)
    return fn(Tbf, src, dst)


def _edge_body(hs_ref, hd_ref, wa_ref, wb_ref, wr2_ref,
               b1_ref, w2_ref, b2_ref, wx_ref, mh_ref, mx_ref):
    ps = hs_ref[...]
    pd = hd_ref[...]
    dx = _unpack_lo(ps) - _unpack_lo(pd)
    r2 = jnp.sum(dx * dx, axis=1, keepdims=True)
    hsh = _unpack_hi(ps)
    hdh = _unpack_hi(pd)
    m = jnp.dot(hsh, wa_ref[...], preferred_element_type=jnp.float32)
    m = m + jnp.dot(hdh, wb_ref[...], preferred_element_type=jnp.float32)
    m = _silu(m + r2 * wr2_ref[...] + b1_ref[...])
    m = _silu(jnp.dot(m, w2_ref[...], preferred_element_type=jnp.float32)
              + b2_ref[...])
    coef = jnp.sum(m * wx_ref[...], axis=1, keepdims=True)
    lane = lax.broadcasted_iota(jnp.int32, (EB, HID), 1)
    mh_ref[...] = m
    mx_ref[...] = dx * coef + (lane == 3).astype(jnp.float32)


def _edge(HS, HD, WA, WB, wr2, b1r, W2, b2r, wxr):
    full = lambda i: (0, 0)
    edge_spec = pl.BlockSpec((EB, HID), lambda i: (i, 0))
    return pl.pallas_call(
        _edge_body,
        grid=(E // EB,),
        in_specs=[
            edge_spec, edge_spec,
            pl.BlockSpec((HID, HID), full),
            pl.BlockSpec((HID, HID), full),
            pl.BlockSpec((1, HID), full),
            pl.BlockSpec((1, HID), full),
            pl.BlockSpec((HID, HID), full),
            pl.BlockSpec((1, HID), full),
            pl.BlockSpec((1, HID), full),
        ],
        out_specs=[edge_spec, edge_spec],
        out_shape=[jax.ShapeDtypeStruct((E, HID), jnp.float32)] * 2,
    )(HS, HD, WA, WB, wr2, b1r, W2, b2r, wxr)


def _scatter_kernel(mh_hbm, mx_hbm, dst_hbm, zeros_hbm, acc_out,
                    idx_v, rows_v, acc_sh, sem):
    c = lax.axis_index("c")
    s = lax.axis_index("s")
    pltpu.sync_copy(zeros_hbm,
                    acc_sh.at[pl.ds(s * ROWS_PER_TILE, ROWS_PER_TILE)])
    plsc.subcore_barrier()

    def add_chunks(m_hbm):
        def body(i, carry):
            ch = s + i * NS

            @pl.when(ch < NCH)
            def _():
                base = ch * CH
                pltpu.sync_copy(dst_hbm.at[pl.ds(base, CH)], idx_v)
                pltpu.sync_copy(m_hbm.at[pl.ds(base, CH)], rows_v)
                pltpu.sync_copy(rows_v, acc_sh.at[idx_v], add=True)

            return carry

        lax.fori_loop(0, SLOOPS, body, 0)

    @pl.when(c == 0)
    def _():
        add_chunks(mh_hbm)

    @pl.when(c == 1)
    def _():
        add_chunks(mx_hbm)

    plsc.subcore_barrier()
    pltpu.sync_copy(acc_sh.at[pl.ds(s * ROWS_PER_TILE, ROWS_PER_TILE)],
                    acc_out.at[c, pl.ds(s * ROWS_PER_TILE, ROWS_PER_TILE)])


def _scatter(MH, MX, dst, zeros_tile):
    fn = functools.partial(
        pl.kernel,
        mesh=_sc_mesh(),
        out_type=jax.ShapeDtypeStruct((NC, N_PAD, HID), jnp.float32),
        scratch_types=[
            pltpu.VMEM((CH,), jnp.int32),
            pltpu.VMEM((CH, HID), jnp.float32),
            pltpu.VMEM_SHARED((N_PAD, HID), jnp.float32),
            pltpu.SemaphoreType.DMA,
        ],
    )(_scatter_kernel)
    return fn(MH, MX, dst, zeros_tile)


def _node_body(th_ref, tx_ref, acc_ref, wua_ref, wub_ref, bu_ref,
               oh_ref, ox_ref, obf_ref):
    h = th_ref[...]
    x = tx_ref[...]
    agg = acc_ref[0]
    xa = acc_ref[1]
    lane = lax.broadcasted_iota(jnp.int32, (NB, HID), 1)
    deg = jnp.sum(jnp.where(lane == 3, xa, 0.0), axis=1, keepdims=True) + 1.0
    u = jnp.dot(h, wua_ref[...], preferred_element_type=jnp.float32)
    u = u + jnp.dot(agg, wub_ref[...], preferred_element_type=jnp.float32)
    hn = h + _silu(u + bu_ref[...])
    xn = x + jnp.where(lane < 3, xa, 0.0) / deg
    oh_ref[...] = hn
    ox_ref[...] = xn
    obf_ref[...] = _pack(hn, xn)


def _node(Th, Tx, ACC, WuA, WuB, bur):
    full = lambda i: (0, 0)
    node_spec = pl.BlockSpec((NB, HID), lambda i: (i, 0))
    return pl.pallas_call(
        _node_body,
        grid=(N // NB,),
        in_specs=[
            node_spec, node_spec,
            pl.BlockSpec((NC, NB, HID), lambda i: (0, i, 0)),
            pl.BlockSpec((HID, HID), full),
            pl.BlockSpec((HID, HID), full),
            pl.BlockSpec((1, HID), full),
        ],
        out_specs=[node_spec, node_spec, node_spec],
        out_shape=[
            jax.ShapeDtypeStruct((N, HID), jnp.float32),
            jax.ShapeDtypeStruct((N, HID), jnp.float32),
            jax.ShapeDtypeStruct((N, HID), jnp.int32),
        ],
    )(Th, Tx, ACC, WuA, WuB, bur)


def _pool_body(th_ref, bid_ref, w1_ref, b1_ref, w2_ref, b2_ref, out_ref):
    bid = bid_ref[...]
    row = lax.broadcasted_iota(jnp.int32, (64, N), 0)
    oht = jnp.where(row == bid, 1.0, 0.0)
    val = jnp.dot(oht, th_ref[...], preferred_element_type=jnp.float32)
    n = jnp.sum(oht, axis=1, keepdims=True)
    val = val * lax.rsqrt(jnp.maximum(n, 1.0))
    o = jnp.dot(_silu(val), w1_ref[...], preferred_element_type=jnp.float32) \
        + b1_ref[...]
    o = jnp.dot(_silu(o), w2_ref[...], preferred_element_type=jnp.float32) \
        + b2_ref[...]
    out_ref[...] = o


def _pool(Th, bid_row, W1, b1r, W2, b2r):
    return pl.pallas_call(
        _pool_body,
        in_specs=[
            pl.BlockSpec((N, HID), lambda: (0, 0)),
            pl.BlockSpec((1, N), lambda: (0, 0)),
            pl.BlockSpec((HID, HID), lambda: (0, 0)),
            pl.BlockSpec((1, HID), lambda: (0, 0)),
            pl.BlockSpec((HID, 128), lambda: (0, 0)),
            pl.BlockSpec((1, 128), lambda: (0, 0)),
        ],
        out_specs=pl.BlockSpec((64, 128), lambda: (0, 0)),
        out_shape=jax.ShapeDtypeStruct((64, 128), jnp.float32),
    )(Th, bid_row, W1, b1r, W2, b2r)


def kernel(H0, Z0, beta, batch_ids, edge_index, W_embed, b_embed, Wm1, bm1,
           Wm2, bm2, Wx, Wu, bu, W1, b1, W2, b2):
    f32 = jnp.float32
    src = edge_index[0].astype(jnp.int32)
    dst = edge_index[1].astype(jnp.int32)
    bid_row = batch_ids.astype(jnp.int32).reshape(1, N)
    beta2 = beta.reshape(N, 1)
    zeros_tile = jnp.zeros((ROWS_PER_TILE, HID), f32)

    z16 = jnp.concatenate([Z0[:, 0, :], jnp.zeros((N, 13), f32)], axis=1)
    Th, Tbf = _embed(H0, beta2, z16, W_embed[:8], W_embed[8:9], W_embed[9:10],
                     W_embed[10:11], b_embed.reshape(1, HID))
    Tx = jnp.concatenate([Z0[:, 0, :], jnp.zeros((N, HID - 3), f32)], axis=1)

    for l in range(3):
        HS, HD = _gather(Tbf, src, dst)
        MH, MX = _edge(HS, HD,
                       Wm1[l, :HID], Wm1[l, HID:2 * HID],
                       Wm1[l, 2 * HID:2 * HID + 1],
                       bm1[l].reshape(1, HID), Wm2[l], bm2[l].reshape(1, HID),
                       Wx[l, :, 0].reshape(1, HID))
        ACC = _scatter(MH, MX, dst, zeros_tile)
        Th, Tx, Tbf = _node(Th, Tx, ACC, Wu[l, :HID], Wu[l, HID:],
                            bu[l].reshape(1, HID))

    return _pool(Th, bid_row, W1, b1.reshape(1, HID), W2, b2.reshape(1, 128))
